# Initial kernel scaffold; baseline (speedup 1.0000x reference)
import functools

import jax
import jax.numpy as jnp
from jax import lax
from jax.experimental import pallas as pl
from jax.experimental.pallas import tpu as pltpu
from jax.experimental.pallas import tpu_sc as plsc

N = 10000
E = 320000
D = 128
NC = 2
NS = 16
NW = NC * NS
EPW = E // NW
C = 80
NCHUNK = EPW // C
GROUPS = C // 16
ZR = 125
RPT = N // NS
S1D = 624
RS = 320
SUB = 80

f32 = jnp.float32
i32 = jnp.int32

_mesh = plsc.VectorSubcoreMesh(
    core_axis_name="c", subcore_axis_name="s", num_cores=NC, num_subcores=NS)


def _zero_rows(ref, nrows):
  def body(r, carry):
    for j in range(D // 16):
      ref[r, pl.ds(j * 16, 16)] = jnp.zeros((16,), f32)
    return carry
  lax.fori_loop(0, nrows, body, 0)


def _fill_1d(ref, n, value):
  def body(g, carry):
    ref[pl.ds(g * 16, 16)] = jnp.full((16,), value, f32)
    return carry
  lax.fori_loop(0, n // 16, body, 0)


def _zero_acc_2d(acc_s, zbuf, sid):
  for k in range(RPT // ZR):
    pltpu.sync_copy(zbuf, acc_s.at[pl.ds(sid * RPT + k * ZR, ZR)])


def _copy_1d_striped(src_ref, dst_ref, sid):
  pltpu.sync_copy(src_ref.at[pl.ds(sid * S1D, S1D)],
                  dst_ref.at[pl.ds(sid * S1D, S1D)])
  @pl.when(sid == NS - 1)
  def _():
    pltpu.sync_copy(src_ref.at[pl.ds(NS * S1D, N - NS * S1D)],
                    dst_ref.at[pl.ds(NS * S1D, N - NS * S1D)])


def _zero_1d_striped(zvec, dst_ref, sid):
  pltpu.sync_copy(zvec.at[pl.ds(0, S1D)], dst_ref.at[pl.ds(sid * S1D, S1D)])
  @pl.when(sid == NS - 1)
  def _():
    pltpu.sync_copy(zvec.at[pl.ds(0, N - NS * S1D)],
                    dst_ref.at[pl.ds(NS * S1D, N - NS * S1D)])


def _k1_body(x_ref, wt_ref, a_ref, h_ref, s12_ref):
  h = jnp.dot(x_ref[...], wt_ref[...], preferred_element_type=f32)
  h_ref[...] = h
  s12_ref[...] = jnp.dot(h, a_ref[...], preferred_element_type=f32)


def _k1(x, wt, a):
  blk = 1000
  grid = (N // blk,)
  return pl.pallas_call(
      _k1_body,
      grid=grid,
      in_specs=[
          pl.BlockSpec((blk, D), lambda i: (i, 0)),
          pl.BlockSpec((D, D), lambda i: (0, 0)),
          pl.BlockSpec((D, 2), lambda i: (0, 0)),
      ],
      out_specs=[
          pl.BlockSpec((blk, D), lambda i: (i, 0)),
          pl.BlockSpec((blk, 2), lambda i: (i, 0)),
      ],
      out_shape=[
          jax.ShapeDtypeStruct((N, D), f32),
          jax.ShapeDtypeStruct((N, 2), f32),
      ],
  )(x, wt, a)


def _ka_body(s12, ei, h, lu_out, den_out, deg_out,
             s12_v, src_v, dst_v, rows_v, w_v, ones_v, zbuf, zvec,
             acc_s, den_s, deg_s, sem):
  cid = lax.axis_index("c")
  sid = lax.axis_index("s")
  wid = sid * NC + cid

  _zero_rows(zbuf, ZR)
  _fill_1d(zvec, S1D + 16, 0.0)
  _fill_1d(ones_v, C, 1.0)
  pltpu.sync_copy(s12, s12_v)
  _zero_acc_2d(acc_s, zbuf, sid)
  _zero_1d_striped(zvec, den_s, sid)
  _zero_1d_striped(zvec, deg_s, sid)
  plsc.subcore_barrier()

  def chunk_body(ci, carry):
    off = wid * EPW + ci * C
    pltpu.sync_copy(ei.at[0, pl.ds(off, C)], src_v)
    pltpu.sync_copy(ei.at[1, pl.ds(off, C)], dst_v)
    pltpu.async_copy(h.at[src_v], rows_v, sem).wait()

    def group_body(g, gcarry):
      b = g * 16
      sv = src_v[pl.ds(b, 16)]
      dv = dst_v[pl.ds(b, 16)]
      s1 = plsc.load_gather(s12_v, [sv, jnp.zeros((16,), i32)])
      s2 = plsc.load_gather(s12_v, [dv, jnp.full((16,), 1, i32)])
      e = s1 + s2
      e = jnp.where(e >= 0.0, e, 0.2 * e)
      w = jnp.exp(e)
      w_v[pl.ds(b, 16)] = w

      def row_body(i, rcarry):
        r = b + i
        wb = plsc.load_gather(w_v, [jnp.full((16,), r, i32)])
        for j in range(D // 16):
          rows_v[r, pl.ds(j * 16, 16)] = rows_v[r, pl.ds(j * 16, 16)] * wb
        return rcarry
      lax.fori_loop(0, 16, row_body, 0)
      return gcarry
    lax.fori_loop(0, GROUPS, group_body, 0)

    pltpu.sync_copy(rows_v, acc_s.at[dst_v], add=True)
    pltpu.sync_copy(w_v, den_s.at[dst_v], add=True)
    pltpu.sync_copy(ones_v, deg_s.at[dst_v], add=True)
    return carry
  lax.fori_loop(0, NCHUNK, chunk_body, 0)

  plsc.subcore_barrier()
  for k in range(RPT // ZR):
    off = sid * RPT + k * ZR
    pltpu.sync_copy(acc_s.at[pl.ds(off, ZR)], lu_out.at[cid, pl.ds(off, ZR)])
  _copy_1d_striped(den_s, den_out.at[cid], sid)
  _copy_1d_striped(deg_s, deg_out.at[cid], sid)


_ka = functools.partial(
    pl.kernel,
    _ka_body,
    out_type=[
        jax.ShapeDtypeStruct((NC, N, D), f32),
        jax.ShapeDtypeStruct((NC, N), f32),
        jax.ShapeDtypeStruct((NC, N), f32),
    ],
    mesh=_mesh,
    scratch_types=[
        pltpu.VMEM((N, 2), f32),
        pltpu.VMEM((C,), i32),
        pltpu.VMEM((C,), i32),
        pltpu.VMEM((C, D), f32),
        pltpu.VMEM((C,), f32),
        pltpu.VMEM((C,), f32),
        pltpu.VMEM((ZR, D), f32),
        pltpu.VMEM((S1D + 16,), f32),
        pltpu.VMEM_SHARED((N, D), f32),
        pltpu.VMEM_SHARED((N,), f32),
        pltpu.VMEM_SHARED((N,), f32),
        pltpu.SemaphoreType.DMA,
    ],
)()


def _kb_body(ei, table, g_out, src_v, dst_v, rows_v, zbuf, acc_s, sem):
  cid = lax.axis_index("c")
  sid = lax.axis_index("s")
  wid = sid * NC + cid

  _zero_rows(zbuf, ZR)
  _zero_acc_2d(acc_s, zbuf, sid)
  plsc.subcore_barrier()

  def chunk_body(ci, carry):
    off = wid * EPW + ci * C
    pltpu.sync_copy(ei.at[0, pl.ds(off, C)], src_v)
    pltpu.sync_copy(ei.at[1, pl.ds(off, C)], dst_v)
    pltpu.async_copy(table.at[src_v], rows_v, sem).wait()
    pltpu.sync_copy(rows_v, acc_s.at[dst_v], add=True)
    return carry
  lax.fori_loop(0, NCHUNK, chunk_body, 0)

  plsc.subcore_barrier()
  for k in range(RPT // ZR):
    off = sid * RPT + k * ZR
    pltpu.sync_copy(acc_s.at[pl.ds(off, ZR)], g_out.at[cid, pl.ds(off, ZR)])


_kb = functools.partial(
    pl.kernel,
    _kb_body,
    out_type=jax.ShapeDtypeStruct((NC, N, D), f32),
    mesh=_mesh,
    scratch_types=[
        pltpu.VMEM((C,), i32),
        pltpu.VMEM((C,), i32),
        pltpu.VMEM((C, D), f32),
        pltpu.VMEM((ZR, D), f32),
        pltpu.VMEM_SHARED((N, D), f32),
        pltpu.SemaphoreType.DMA,
    ],
)()


def _kc_body(g1u, den, deg, g1_out, invdeg_out, invden_out,
             buf0, buf1, d0_v, d1_v, n0_v, n1_v, invdeg_v, invden_v, sem):
  cid = lax.axis_index("c")
  sid = lax.axis_index("s")
  wid = sid * NC + cid

  for k in range(RS // SUB):
    row0 = wid * RS + k * SUB

    @pl.when(row0 < N)
    def _():
      pltpu.sync_copy(g1u.at[0, pl.ds(row0, SUB)], buf0)
      pltpu.sync_copy(g1u.at[1, pl.ds(row0, SUB)], buf1)
      pltpu.sync_copy(deg.at[0, pl.ds(row0, SUB)], d0_v)
      pltpu.sync_copy(deg.at[1, pl.ds(row0, SUB)], d1_v)
      pltpu.sync_copy(den.at[0, pl.ds(row0, SUB)], n0_v)
      pltpu.sync_copy(den.at[1, pl.ds(row0, SUB)], n1_v)

      def vec_body(g, carry):
        b = g * 16
        dg = d0_v[pl.ds(b, 16)] + d1_v[pl.ds(b, 16)]
        dg = jnp.maximum(dg, 1.0)
        invdeg_v[pl.ds(b, 16)] = 1.0 / dg
        dn = n0_v[pl.ds(b, 16)] + n1_v[pl.ds(b, 16)] + 1e-16
        invden_v[pl.ds(b, 16)] = 1.0 / dn
        return carry
      lax.fori_loop(0, SUB // 16, vec_body, 0)

      def row_body(i, carry):
        wb = plsc.load_gather(invdeg_v, [jnp.full((16,), i, i32)])
        for j in range(D // 16):
          buf0[i, pl.ds(j * 16, 16)] = (
              buf0[i, pl.ds(j * 16, 16)] + buf1[i, pl.ds(j * 16, 16)]) * wb
        return carry
      lax.fori_loop(0, SUB, row_body, 0)

      pltpu.sync_copy(buf0, g1_out.at[pl.ds(row0, SUB)])
      pltpu.sync_copy(invdeg_v, invdeg_out.at[pl.ds(row0, SUB)])
      pltpu.sync_copy(invden_v, invden_out.at[pl.ds(row0, SUB)])


_kc = functools.partial(
    pl.kernel,
    _kc_body,
    out_type=[
        jax.ShapeDtypeStruct((N, D), f32),
        jax.ShapeDtypeStruct((N,), f32),
        jax.ShapeDtypeStruct((N,), f32),
    ],
    mesh=_mesh,
    scratch_types=[
        pltpu.VMEM((SUB, D), f32),
        pltpu.VMEM((SUB, D), f32),
        pltpu.VMEM((SUB,), f32),
        pltpu.VMEM((SUB,), f32),
        pltpu.VMEM((SUB,), f32),
        pltpu.VMEM((SUB,), f32),
        pltpu.VMEM((SUB,), f32),
        pltpu.VMEM((SUB,), f32),
        pltpu.SemaphoreType.DMA,
    ],
)()


def _kf_body(lu0_ref, lu1_ref, invden_ref, g2u0_ref, g2u1_ref, invdeg_ref,
             gftwt_ref, w1t_ref, w2t_ref, gb_ref, bb_ref, out_ref):
  lu = (lu0_ref[...] + lu1_ref[...]) * invden_ref[...]
  local = jnp.where(lu > 0.0, lu, jnp.expm1(lu))
  g2 = (g2u0_ref[...] + g2u1_ref[...]) * invdeg_ref[...]
  gf = jnp.dot(g2, gftwt_ref[...], preferred_element_type=f32) + gb_ref[...]
  acc = jnp.dot(local, w1t_ref[...], preferred_element_type=f32)
  acc = acc + jnp.dot(gf, w2t_ref[...], preferred_element_type=f32)
  out_ref[...] = jnp.maximum(acc + bb_ref[...], 0.0)


def _kf(lu0, lu1, invden, g2u0, g2u1, invdeg, gftwt, w1t, w2t, gb, bb):
  blk = 1000
  grid = (N // blk,)
  big = pl.BlockSpec((blk, D), lambda i: (i, 0))
  col = pl.BlockSpec((blk, 1), lambda i: (i, 0))
  wgt = pl.BlockSpec((D, D), lambda i: (0, 0))
  row = pl.BlockSpec((1, D), lambda i: (0, 0))
  return pl.pallas_call(
      _kf_body,
      grid=grid,
      in_specs=[big, big, col, big, big, col, wgt, wgt, wgt, row, row],
      out_specs=big,
      out_shape=jax.ShapeDtypeStruct((N, D), f32),
  )(lu0, lu1, invden, g2u0, g2u1, invdeg, gftwt, w1t, w2t, gb, bb)


@jax.jit
def kernel(node_features, edge_index, linear_weights, attention_weights,
           wt_W, wt_b, gft_W, gft_b):
  wt = linear_weights.T
  a = jnp.reshape(attention_weights, (2, D)).T
  h, s12 = _k1(node_features, wt, a)

  lu, den, deg = _ka(s12, edge_index, h)
  g1u = _kb(edge_index, node_features)
  g1, invdeg, invden = _kc(g1u, den, deg)
  g2u = _kb(edge_index, g1)

  out = _kf(lu[0], lu[1], invden.reshape(N, 1),
            g2u[0], g2u[1], invdeg.reshape(N, 1),
            gft_W.T, wt_W[:, :D].T, wt_W[:, D:].T,
            gft_b.reshape(1, D), wt_b.reshape(1, D))
  return out


# speedup vs baseline: 7.3369x; 7.3369x over previous
import functools

import jax
import jax.numpy as jnp
from jax import lax
from jax.experimental import pallas as pl
from jax.experimental.pallas import tpu as pltpu
from jax.experimental.pallas import tpu_sc as plsc

N = 10000
E = 320000
D = 128
NC = 2
NS = 16
NW = NC * NS
EPW = E // NW
C = 80
NCHUNK = EPW // C
GROUPS = C // 16
ZR = 125
RPT = N // NS
S1D = 624
RS = 320
SUB = 80

f32 = jnp.float32
i32 = jnp.int32

_mesh = plsc.VectorSubcoreMesh(
    core_axis_name="c", subcore_axis_name="s", num_cores=NC, num_subcores=NS)


def _zero_rows(ref, nrows):
  def body(r, carry):
    for j in range(D // 16):
      ref[r, pl.ds(j * 16, 16)] = jnp.zeros((16,), f32)
    return carry
  lax.fori_loop(0, nrows, body, 0)


def _fill_1d(ref, n, value):
  def body(g, carry):
    ref[pl.ds(g * 16, 16)] = jnp.full((16,), value, f32)
    return carry
  lax.fori_loop(0, n // 16, body, 0)


def _zero_acc_2d(acc_s, zbuf, sid):
  for k in range(RPT // ZR):
    pltpu.sync_copy(zbuf, acc_s.at[pl.ds(sid * RPT + k * ZR, ZR)])


def _copy_1d_striped(src_ref, dst_ref, sid):
  pltpu.sync_copy(src_ref.at[pl.ds(sid * S1D, S1D)],
                  dst_ref.at[pl.ds(sid * S1D, S1D)])
  @pl.when(sid == NS - 1)
  def _():
    pltpu.sync_copy(src_ref.at[pl.ds(NS * S1D, N - NS * S1D)],
                    dst_ref.at[pl.ds(NS * S1D, N - NS * S1D)])


def _zero_1d_striped(zvec, dst_ref, sid):
  pltpu.sync_copy(zvec.at[pl.ds(0, S1D)], dst_ref.at[pl.ds(sid * S1D, S1D)])
  @pl.when(sid == NS - 1)
  def _():
    pltpu.sync_copy(zvec.at[pl.ds(0, N - NS * S1D)],
                    dst_ref.at[pl.ds(NS * S1D, N - NS * S1D)])


def _k1_body(x_ref, wt_ref, a_ref, h_ref, s12_ref):
  h = jnp.dot(x_ref[...], wt_ref[...], preferred_element_type=f32)
  h_ref[...] = h
  s12_ref[...] = jnp.dot(h, a_ref[...], preferred_element_type=f32)


def _k1(x, wt, a):
  blk = 1000
  grid = (N // blk,)
  return pl.pallas_call(
      _k1_body,
      grid=grid,
      in_specs=[
          pl.BlockSpec((blk, D), lambda i: (i, 0)),
          pl.BlockSpec((D, D), lambda i: (0, 0)),
          pl.BlockSpec((D, 2), lambda i: (0, 0)),
      ],
      out_specs=[
          pl.BlockSpec((blk, D), lambda i: (i, 0)),
          pl.BlockSpec((blk, 2), lambda i: (i, 0)),
      ],
      out_shape=[
          jax.ShapeDtypeStruct((N, D), f32),
          jax.ShapeDtypeStruct((N, 2), f32),
      ],
  )(x, wt, a)


def _ka_body(s12, ei, h, lu_out, den_out, deg_out,
             s12_v, src_v, dst_v, rows_v, w_v, ones_v, zbuf, zvec,
             acc_s, den_s, deg_s, sem):
  cid = lax.axis_index("c")
  sid = lax.axis_index("s")
  wid = sid * NC + cid

  _zero_rows(zbuf, ZR)
  _fill_1d(zvec, S1D + 16, 0.0)
  _fill_1d(ones_v, C, 1.0)
  pltpu.sync_copy(s12, s12_v)
  _zero_acc_2d(acc_s, zbuf, sid)
  _zero_1d_striped(zvec, den_s, sid)
  _zero_1d_striped(zvec, deg_s, sid)
  plsc.subcore_barrier()

  def chunk_body(ci, carry):
    off = wid * EPW + ci * C
    pltpu.sync_copy(ei.at[0, pl.ds(off, C)], src_v)
    pltpu.sync_copy(ei.at[1, pl.ds(off, C)], dst_v)
    pltpu.async_copy(h.at[src_v], rows_v, sem).wait()

    def group_body(g, gcarry):
      b = g * 16
      sv = src_v[pl.ds(b, 16)]
      dv = dst_v[pl.ds(b, 16)]
      s1 = plsc.load_gather(s12_v, [sv * 2])
      s2 = plsc.load_gather(s12_v, [dv * 2 + 1])
      e = s1 + s2
      e = jnp.where(e >= 0.0, e, 0.2 * e)
      w = jnp.exp(e)
      w_v[pl.ds(b, 16)] = w

      def row_body(i, rcarry):
        r = b + i
        wb = plsc.load_gather(w_v, [jnp.full((16,), r, i32)])
        for j in range(D // 16):
          rows_v[r, pl.ds(j * 16, 16)] = rows_v[r, pl.ds(j * 16, 16)] * wb
        return rcarry
      lax.fori_loop(0, 16, row_body, 0)
      return gcarry
    lax.fori_loop(0, GROUPS, group_body, 0)

    pltpu.sync_copy(rows_v, acc_s.at[dst_v], add=True)
    pltpu.sync_copy(w_v, den_s.at[dst_v], add=True)
    pltpu.sync_copy(ones_v, deg_s.at[dst_v], add=True)
    return carry
  lax.fori_loop(0, NCHUNK, chunk_body, 0)

  plsc.subcore_barrier()
  for k in range(RPT // ZR):
    off = sid * RPT + k * ZR
    pltpu.sync_copy(acc_s.at[pl.ds(off, ZR)], lu_out.at[cid, pl.ds(off, ZR)])
  _copy_1d_striped(den_s, den_out.at[cid], sid)
  _copy_1d_striped(deg_s, deg_out.at[cid], sid)


_ka = functools.partial(
    pl.kernel,
    _ka_body,
    out_type=[
        jax.ShapeDtypeStruct((NC, N, D), f32),
        jax.ShapeDtypeStruct((NC, N), f32),
        jax.ShapeDtypeStruct((NC, N), f32),
    ],
    mesh=_mesh,
    compiler_params=pltpu.CompilerParams(use_tc_tiling_on_sc=False, needs_layout_passes=False),
    scratch_types=[
        pltpu.VMEM((2 * N,), f32),
        pltpu.VMEM((C,), i32),
        pltpu.VMEM((C,), i32),
        pltpu.VMEM((C, D), f32),
        pltpu.VMEM((C,), f32),
        pltpu.VMEM((C,), f32),
        pltpu.VMEM((ZR, D), f32),
        pltpu.VMEM((S1D + 16,), f32),
        pltpu.VMEM_SHARED((N, D), f32),
        pltpu.VMEM_SHARED((N,), f32),
        pltpu.VMEM_SHARED((N,), f32),
        pltpu.SemaphoreType.DMA,
    ],
)()


def _kb_body(ei, table, g_out, src_v, dst_v, rows_v, zbuf, acc_s, sem):
  cid = lax.axis_index("c")
  sid = lax.axis_index("s")
  wid = sid * NC + cid

  _zero_rows(zbuf, ZR)
  _zero_acc_2d(acc_s, zbuf, sid)
  plsc.subcore_barrier()

  def chunk_body(ci, carry):
    off = wid * EPW + ci * C
    pltpu.sync_copy(ei.at[0, pl.ds(off, C)], src_v)
    pltpu.sync_copy(ei.at[1, pl.ds(off, C)], dst_v)
    pltpu.async_copy(table.at[src_v], rows_v, sem).wait()
    pltpu.sync_copy(rows_v, acc_s.at[dst_v], add=True)
    return carry
  lax.fori_loop(0, NCHUNK, chunk_body, 0)

  plsc.subcore_barrier()
  for k in range(RPT // ZR):
    off = sid * RPT + k * ZR
    pltpu.sync_copy(acc_s.at[pl.ds(off, ZR)], g_out.at[cid, pl.ds(off, ZR)])


_kb = functools.partial(
    pl.kernel,
    _kb_body,
    out_type=jax.ShapeDtypeStruct((NC, N, D), f32),
    mesh=_mesh,
    compiler_params=pltpu.CompilerParams(use_tc_tiling_on_sc=False, needs_layout_passes=False),
    scratch_types=[
        pltpu.VMEM((C,), i32),
        pltpu.VMEM((C,), i32),
        pltpu.VMEM((C, D), f32),
        pltpu.VMEM((ZR, D), f32),
        pltpu.VMEM_SHARED((N, D), f32),
        pltpu.SemaphoreType.DMA,
    ],
)()


def _kc_body(g1u, den, deg, g1_out, invdeg_out, invden_out,
             buf0, buf1, d0_v, d1_v, n0_v, n1_v, invdeg_v, invden_v, sem):
  cid = lax.axis_index("c")
  sid = lax.axis_index("s")
  wid = sid * NC + cid

  for k in range(RS // SUB):
    row0 = wid * RS + k * SUB

    @pl.when(row0 < N)
    def _():
      pltpu.sync_copy(g1u.at[0, pl.ds(row0, SUB)], buf0)
      pltpu.sync_copy(g1u.at[1, pl.ds(row0, SUB)], buf1)
      pltpu.sync_copy(deg.at[0, pl.ds(row0, SUB)], d0_v)
      pltpu.sync_copy(deg.at[1, pl.ds(row0, SUB)], d1_v)
      pltpu.sync_copy(den.at[0, pl.ds(row0, SUB)], n0_v)
      pltpu.sync_copy(den.at[1, pl.ds(row0, SUB)], n1_v)

      def vec_body(g, carry):
        b = g * 16
        dg = d0_v[pl.ds(b, 16)] + d1_v[pl.ds(b, 16)]
        dg = jnp.maximum(dg, 1.0)
        invdeg_v[pl.ds(b, 16)] = 1.0 / dg
        dn = n0_v[pl.ds(b, 16)] + n1_v[pl.ds(b, 16)] + 1e-16
        invden_v[pl.ds(b, 16)] = 1.0 / dn
        return carry
      lax.fori_loop(0, SUB // 16, vec_body, 0)

      def row_body(i, carry):
        wb = plsc.load_gather(invdeg_v, [jnp.full((16,), i, i32)])
        for j in range(D // 16):
          buf0[i, pl.ds(j * 16, 16)] = (
              buf0[i, pl.ds(j * 16, 16)] + buf1[i, pl.ds(j * 16, 16)]) * wb
        return carry
      lax.fori_loop(0, SUB, row_body, 0)

      pltpu.sync_copy(buf0, g1_out.at[pl.ds(row0, SUB)])
      pltpu.sync_copy(invdeg_v, invdeg_out.at[pl.ds(row0, SUB)])
      pltpu.sync_copy(invden_v, invden_out.at[pl.ds(row0, SUB)])


_kc = functools.partial(
    pl.kernel,
    _kc_body,
    out_type=[
        jax.ShapeDtypeStruct((N, D), f32),
        jax.ShapeDtypeStruct((N,), f32),
        jax.ShapeDtypeStruct((N,), f32),
    ],
    mesh=_mesh,
    compiler_params=pltpu.CompilerParams(use_tc_tiling_on_sc=False, needs_layout_passes=False),
    scratch_types=[
        pltpu.VMEM((SUB, D), f32),
        pltpu.VMEM((SUB, D), f32),
        pltpu.VMEM((SUB,), f32),
        pltpu.VMEM((SUB,), f32),
        pltpu.VMEM((SUB,), f32),
        pltpu.VMEM((SUB,), f32),
        pltpu.VMEM((SUB,), f32),
        pltpu.VMEM((SUB,), f32),
        pltpu.SemaphoreType.DMA,
    ],
)()


def _kf_body(lu0_ref, lu1_ref, invden_ref, g2u0_ref, g2u1_ref, invdeg_ref,
             gftwt_ref, w1t_ref, w2t_ref, gb_ref, bb_ref, out_ref):
  lu = (lu0_ref[...] + lu1_ref[...]) * invden_ref[...]
  local = jnp.where(lu > 0.0, lu, jnp.exp(jnp.minimum(lu, 0.0)) - 1.0)
  g2 = (g2u0_ref[...] + g2u1_ref[...]) * invdeg_ref[...]
  gf = jnp.dot(g2, gftwt_ref[...], preferred_element_type=f32) + gb_ref[...]
  acc = jnp.dot(local, w1t_ref[...], preferred_element_type=f32)
  acc = acc + jnp.dot(gf, w2t_ref[...], preferred_element_type=f32)
  out_ref[...] = jnp.maximum(acc + bb_ref[...], 0.0)


def _kf(lu0, lu1, invden, g2u0, g2u1, invdeg, gftwt, w1t, w2t, gb, bb):
  blk = 1000
  grid = (N // blk,)
  big = pl.BlockSpec((blk, D), lambda i: (i, 0))
  col = pl.BlockSpec((blk, 1), lambda i: (i, 0))
  wgt = pl.BlockSpec((D, D), lambda i: (0, 0))
  row = pl.BlockSpec((1, D), lambda i: (0, 0))
  return pl.pallas_call(
      _kf_body,
      grid=grid,
      in_specs=[big, big, col, big, big, col, wgt, wgt, wgt, row, row],
      out_specs=big,
      out_shape=jax.ShapeDtypeStruct((N, D), f32),
  )(lu0, lu1, invden, g2u0, g2u1, invdeg, gftwt, w1t, w2t, gb, bb)


@jax.jit
def kernel(node_features, edge_index, linear_weights, attention_weights,
           wt_W, wt_b, gft_W, gft_b):
  wt = linear_weights.T
  a = jnp.reshape(attention_weights, (2, D)).T
  h, s12 = _k1(node_features, wt, a)

  lu, den, deg = _ka(s12.reshape(2 * N), edge_index, h)
  g1u = _kb(edge_index, node_features)
  g1, invdeg, invden = _kc(g1u, den, deg)
  g2u = _kb(edge_index, g1)

  out = _kf(lu[0], lu[1], invden.reshape(N, 1),
            g2u[0], g2u[1], invdeg.reshape(N, 1),
            gft_W.T, wt_W[:, :D].T, wt_W[:, D:].T,
            gft_b.reshape(1, D), wt_b.reshape(1, D))
  return out


# speedup vs baseline: 13.0981x; 1.7852x over previous
import functools

import jax
import jax.numpy as jnp
from jax import lax
from jax.experimental import pallas as pl
from jax.experimental.pallas import tpu as pltpu
from jax.experimental.pallas import tpu_sc as plsc

N = 10000
E = 320000
D = 128
NC = 2
NS = 16
NW = NC * NS
EPW = E // NW
C = 80
NCHUNK = EPW // C
GROUPS = C // 16
ZR = 125
RPT = N // NS
S1D = 624
RS = 320
SUB = 80

f32 = jnp.float32
i32 = jnp.int32

_mesh = plsc.VectorSubcoreMesh(
    core_axis_name="c", subcore_axis_name="s", num_cores=NC, num_subcores=NS)


def _zero_rows(ref, nrows):
  def body(r, carry):
    for j in range(D // 16):
      ref[r, pl.ds(j * 16, 16)] = jnp.zeros((16,), f32)
    return carry
  lax.fori_loop(0, nrows, body, 0)


def _fill_1d(ref, n, value):
  def body(g, carry):
    ref[pl.ds(g * 16, 16)] = jnp.full((16,), value, f32)
    return carry
  lax.fori_loop(0, n // 16, body, 0)


def _zero_acc_2d(acc_s, zbuf, sid):
  for k in range(RPT // ZR):
    pltpu.sync_copy(zbuf, acc_s.at[pl.ds(sid * RPT + k * ZR, ZR)])


def _copy_1d_striped(src_ref, dst_ref, sid):
  pltpu.sync_copy(src_ref.at[pl.ds(sid * S1D, S1D)],
                  dst_ref.at[pl.ds(sid * S1D, S1D)])
  @pl.when(sid == NS - 1)
  def _():
    pltpu.sync_copy(src_ref.at[pl.ds(NS * S1D, N - NS * S1D)],
                    dst_ref.at[pl.ds(NS * S1D, N - NS * S1D)])


def _zero_1d_striped(zvec, dst_ref, sid):
  pltpu.sync_copy(zvec.at[pl.ds(0, S1D)], dst_ref.at[pl.ds(sid * S1D, S1D)])
  @pl.when(sid == NS - 1)
  def _():
    pltpu.sync_copy(zvec.at[pl.ds(0, N - NS * S1D)],
                    dst_ref.at[pl.ds(NS * S1D, N - NS * S1D)])


def _k1_body(x_ref, wt_ref, a_ref, h_ref, s12_ref):
  h = jnp.dot(x_ref[...], wt_ref[...], preferred_element_type=f32)
  h_ref[...] = h
  s12_ref[...] = jnp.dot(h, a_ref[...], preferred_element_type=f32)


def _k1(x, wt, a):
  blk = 1000
  grid = (N // blk,)
  return pl.pallas_call(
      _k1_body,
      grid=grid,
      in_specs=[
          pl.BlockSpec((blk, D), lambda i: (i, 0)),
          pl.BlockSpec((D, D), lambda i: (0, 0)),
          pl.BlockSpec((D, 2), lambda i: (0, 0)),
      ],
      out_specs=[
          pl.BlockSpec((blk, D), lambda i: (i, 0)),
          pl.BlockSpec((blk, 2), lambda i: (i, 0)),
      ],
      out_shape=[
          jax.ShapeDtypeStruct((N, D), f32),
          jax.ShapeDtypeStruct((N, 2), f32),
      ],
  )(x, wt, a)


def _ka_copy_idx(dst_s, dstall_v, c):
  for g in range(GROUPS):
    dst_s[pl.ds(g * 16, 16)] = dstall_v[pl.ds(c * C + g * 16, 16)]


def _zero_acc_from_rows(acc_s, rows_v, sid):
  for k in range(RPT // C):
    pltpu.sync_copy(rows_v, acc_s.at[pl.ds(sid * RPT + k * C, C)])
  rem = RPT - (RPT // C) * C
  if rem:
    pltpu.sync_copy(rows_v.at[pl.ds(0, rem)],
                    acc_s.at[pl.ds(sid * RPT + (RPT // C) * C, rem)])


def _ka_body(s1, s2, ei, h, lu_out, den_out, deg_out,
             srcall_v, dstall_v, dstA, dstB, rowsA, rowsB,
             s1A, s1B, s2A, s2B, wA, wB, ones_v, zvec,
             acc_s, den_s, deg_s,
             gsemA, gsemB, ssemA, ssemB):
  cid = lax.axis_index("c")
  sid = lax.axis_index("s")
  wid = sid * NC + cid
  base = wid * EPW

  _zero_rows(rowsA, C)
  _fill_1d(zvec, S1D + 16, 0.0)
  _fill_1d(ones_v, C, 1.0)
  pltpu.sync_copy(ei.at[0, pl.ds(base, EPW)], srcall_v)
  pltpu.sync_copy(ei.at[1, pl.ds(base, EPW)], dstall_v)
  _zero_acc_from_rows(acc_s, rowsA, sid)
  _zero_1d_striped(zvec, den_s, sid)
  _zero_1d_striped(zvec, deg_s, sid)
  plsc.subcore_barrier()

  slots = ((dstA, rowsA, s1A, s2A, wA, gsemA, ssemA),
           (dstB, rowsB, s1B, s2B, wB, gsemB, ssemB))

  def issue(c, k2, slot):
    dstS, rowsS, s1S, s2S, wS, gsem, ssem = slot
    @pl.when(k2 > 0)
    def _():
      pltpu.make_async_copy(rowsS, acc_s.at[dstS], ssem).wait()
    _ka_copy_idx(dstS, dstall_v, c)
    pltpu.async_copy(h.at[srcall_v.at[pl.ds(c * C, C)]], rowsS, gsem)
    pltpu.async_copy(s1.at[srcall_v.at[pl.ds(c * C, C)]], s1S, gsem)
    pltpu.async_copy(s2.at[dstall_v.at[pl.ds(c * C, C)]], s2S, gsem)

  def process(c, slot):
    dstS, rowsS, s1S, s2S, wS, gsem, ssem = slot
    pltpu.make_async_copy(h.at[srcall_v.at[pl.ds(c * C, C)]], rowsS, gsem
                          ).wait()
    pltpu.make_async_copy(s1.at[srcall_v.at[pl.ds(c * C, C)]], s1S, gsem
                          ).wait()
    pltpu.make_async_copy(s2.at[dstall_v.at[pl.ds(c * C, C)]], s2S, gsem
                          ).wait()

    def group_body(g, gcarry):
      b = g * 16
      e = s1S[pl.ds(b, 16)] + s2S[pl.ds(b, 16)]
      e = jnp.where(e >= 0.0, e, 0.2 * e)
      w = jnp.exp(e)
      wS[pl.ds(b, 16)] = w
      for i in range(16):
        r = b + i
        wb = plsc.load_gather(wS, [jnp.full((16,), r, i32)])
        for j in range(D // 16):
          rows_slice = rowsS[r, pl.ds(j * 16, 16)]
          rowsS[r, pl.ds(j * 16, 16)] = rows_slice * wb
      return gcarry
    lax.fori_loop(0, GROUPS, group_body, 0)

    pltpu.async_copy(rowsS, acc_s.at[dstS], ssem, add=True)
    pltpu.sync_copy(wS, den_s.at[dstS], add=True)
    pltpu.sync_copy(ones_v, deg_s.at[dstS], add=True)

  def body2(k2, carry):
    cA = 2 * k2
    cB = cA + 1
    @pl.when(cA < NCHUNK)
    def _():
      issue(cA, k2, slots[0])
    @pl.when(cB < NCHUNK)
    def _():
      issue(cB, k2, slots[1])
    @pl.when(cA < NCHUNK)
    def _():
      process(cA, slots[0])
    @pl.when(cB < NCHUNK)
    def _():
      process(cB, slots[1])
    return carry
  lax.fori_loop(0, (NCHUNK + 1) // 2, body2, 0)
  pltpu.make_async_copy(rowsA, acc_s.at[dstA], ssemA).wait()
  pltpu.make_async_copy(rowsB, acc_s.at[dstB], ssemB).wait()

  plsc.subcore_barrier()
  for k in range(RPT // ZR):
    off = sid * RPT + k * ZR
    pltpu.sync_copy(acc_s.at[pl.ds(off, ZR)], lu_out.at[cid, pl.ds(off, ZR)])
  _copy_1d_striped(den_s, den_out.at[cid], sid)
  _copy_1d_striped(deg_s, deg_out.at[cid], sid)


_ka = functools.partial(
    pl.kernel,
    _ka_body,
    out_type=[
        jax.ShapeDtypeStruct((NC, N, D), f32),
        jax.ShapeDtypeStruct((NC, N), f32),
        jax.ShapeDtypeStruct((NC, N), f32),
    ],
    mesh=_mesh,
    compiler_params=pltpu.CompilerParams(use_tc_tiling_on_sc=False, needs_layout_passes=False),
    scratch_types=[
        pltpu.VMEM((EPW,), i32),
        pltpu.VMEM((EPW,), i32),
        pltpu.VMEM((C,), i32),
        pltpu.VMEM((C,), i32),
        pltpu.VMEM((C, D), f32),
        pltpu.VMEM((C, D), f32),
        pltpu.VMEM((C,), f32),
        pltpu.VMEM((C,), f32),
        pltpu.VMEM((C,), f32),
        pltpu.VMEM((C,), f32),
        pltpu.VMEM((C,), f32),
        pltpu.VMEM((C,), f32),
        pltpu.VMEM((C,), f32),
        pltpu.VMEM((S1D + 16,), f32),
        pltpu.VMEM_SHARED((N, D), f32),
        pltpu.VMEM_SHARED((N,), f32),
        pltpu.VMEM_SHARED((N,), f32),
        pltpu.SemaphoreType.DMA,
        pltpu.SemaphoreType.DMA,
        pltpu.SemaphoreType.DMA,
        pltpu.SemaphoreType.DMA,
    ],
)()


def _kb_body(ei, table, tok, g_out,
             srcall_v, dstall_v, dstA, dstB, rowsA, rowsB, acc_s,
             gsemA, gsemB, ssemA, ssemB):
  cid = lax.axis_index("c")
  sid = lax.axis_index("s")
  wid = sid * NC + cid
  base = wid * EPW

  _zero_rows(rowsA, C)
  pltpu.sync_copy(ei.at[0, pl.ds(base, EPW)], srcall_v)
  pltpu.sync_copy(ei.at[1, pl.ds(base, EPW)], dstall_v)
  _zero_acc_from_rows(acc_s, rowsA, sid)
  plsc.subcore_barrier()

  slots = ((dstA, rowsA, gsemA, ssemA), (dstB, rowsB, gsemB, ssemB))

  def issue(c, k2, slot):
    dstS, rowsS, gsem, ssem = slot
    @pl.when(k2 > 0)
    def _():
      pltpu.make_async_copy(rowsS, acc_s.at[dstS], ssem).wait()
    _ka_copy_idx(dstS, dstall_v, c)
    pltpu.async_copy(table.at[srcall_v.at[pl.ds(c * C, C)]], rowsS, gsem)

  def process(c, slot):
    dstS, rowsS, gsem, ssem = slot
    pltpu.make_async_copy(table.at[srcall_v.at[pl.ds(c * C, C)]], rowsS, gsem
                          ).wait()
    pltpu.async_copy(rowsS, acc_s.at[dstS], ssem, add=True)

  def body2(k2, carry):
    cA = 2 * k2
    cB = cA + 1
    @pl.when(cA < NCHUNK)
    def _():
      issue(cA, k2, slots[0])
    @pl.when(cB < NCHUNK)
    def _():
      issue(cB, k2, slots[1])
    @pl.when(cA < NCHUNK)
    def _():
      process(cA, slots[0])
    @pl.when(cB < NCHUNK)
    def _():
      process(cB, slots[1])
    return carry
  lax.fori_loop(0, (NCHUNK + 1) // 2, body2, 0)
  pltpu.make_async_copy(rowsA, acc_s.at[dstA], ssemA).wait()
  pltpu.make_async_copy(rowsB, acc_s.at[dstB], ssemB).wait()

  plsc.subcore_barrier()
  for k in range(RPT // ZR):
    off = sid * RPT + k * ZR
    pltpu.sync_copy(acc_s.at[pl.ds(off, ZR)], g_out.at[cid, pl.ds(off, ZR)])


_kb = functools.partial(
    pl.kernel,
    _kb_body,
    out_type=jax.ShapeDtypeStruct((NC, N, D), f32),
    mesh=_mesh,
    compiler_params=pltpu.CompilerParams(use_tc_tiling_on_sc=False, needs_layout_passes=False),
    scratch_types=[
        pltpu.VMEM((EPW,), i32),
        pltpu.VMEM((EPW,), i32),
        pltpu.VMEM((C,), i32),
        pltpu.VMEM((C,), i32),
        pltpu.VMEM((C, D), f32),
        pltpu.VMEM((C, D), f32),
        pltpu.VMEM_SHARED((N, D), f32),
        pltpu.SemaphoreType.DMA,
        pltpu.SemaphoreType.DMA,
        pltpu.SemaphoreType.DMA,
        pltpu.SemaphoreType.DMA,
    ],
)()


def _kc_body(g1u, den, deg, g1_out, invdeg_out, invden_out,
             buf0, buf1, d0_v, d1_v, n0_v, n1_v, invdeg_v, invden_v, sem):
  cid = lax.axis_index("c")
  sid = lax.axis_index("s")
  wid = sid * NC + cid

  for k in range(RS // SUB):
    row0 = wid * RS + k * SUB

    @pl.when(row0 < N)
    def _():
      pltpu.sync_copy(g1u.at[0, pl.ds(row0, SUB)], buf0)
      pltpu.sync_copy(g1u.at[1, pl.ds(row0, SUB)], buf1)
      pltpu.sync_copy(deg.at[0, pl.ds(row0, SUB)], d0_v)
      pltpu.sync_copy(deg.at[1, pl.ds(row0, SUB)], d1_v)
      pltpu.sync_copy(den.at[0, pl.ds(row0, SUB)], n0_v)
      pltpu.sync_copy(den.at[1, pl.ds(row0, SUB)], n1_v)

      def vec_body(g, carry):
        b = g * 16
        dg = d0_v[pl.ds(b, 16)] + d1_v[pl.ds(b, 16)]
        dg = jnp.maximum(dg, 1.0)
        invdeg_v[pl.ds(b, 16)] = 1.0 / dg
        dn = n0_v[pl.ds(b, 16)] + n1_v[pl.ds(b, 16)] + 1e-16
        invden_v[pl.ds(b, 16)] = 1.0 / dn
        return carry
      lax.fori_loop(0, SUB // 16, vec_body, 0)

      def row_body(i, carry):
        wb = plsc.load_gather(invdeg_v, [jnp.full((16,), i, i32)])
        for j in range(D // 16):
          buf0[i, pl.ds(j * 16, 16)] = (
              buf0[i, pl.ds(j * 16, 16)] + buf1[i, pl.ds(j * 16, 16)]) * wb
        return carry
      lax.fori_loop(0, SUB, row_body, 0)

      pltpu.sync_copy(buf0, g1_out.at[pl.ds(row0, SUB)])
      pltpu.sync_copy(invdeg_v, invdeg_out.at[pl.ds(row0, SUB)])
      pltpu.sync_copy(invden_v, invden_out.at[pl.ds(row0, SUB)])


_kc = functools.partial(
    pl.kernel,
    _kc_body,
    out_type=[
        jax.ShapeDtypeStruct((N, D), f32),
        jax.ShapeDtypeStruct((N,), f32),
        jax.ShapeDtypeStruct((N,), f32),
    ],
    mesh=_mesh,
    compiler_params=pltpu.CompilerParams(use_tc_tiling_on_sc=False, needs_layout_passes=False),
    scratch_types=[
        pltpu.VMEM((SUB, D), f32),
        pltpu.VMEM((SUB, D), f32),
        pltpu.VMEM((SUB,), f32),
        pltpu.VMEM((SUB,), f32),
        pltpu.VMEM((SUB,), f32),
        pltpu.VMEM((SUB,), f32),
        pltpu.VMEM((SUB,), f32),
        pltpu.VMEM((SUB,), f32),
        pltpu.SemaphoreType.DMA,
    ],
)()


def _kf_body(lu0_ref, lu1_ref, invden_ref, g2u0_ref, g2u1_ref, invdeg_ref,
             gftwt_ref, w1t_ref, w2t_ref, gb_ref, bb_ref, out_ref):
  lu = (lu0_ref[...] + lu1_ref[...]) * invden_ref[...]
  local = jnp.where(lu > 0.0, lu, jnp.exp(jnp.minimum(lu, 0.0)) - 1.0)
  g2 = (g2u0_ref[...] + g2u1_ref[...]) * invdeg_ref[...]
  gf = jnp.dot(g2, gftwt_ref[...], preferred_element_type=f32) + gb_ref[...]
  acc = jnp.dot(local, w1t_ref[...], preferred_element_type=f32)
  acc = acc + jnp.dot(gf, w2t_ref[...], preferred_element_type=f32)
  out_ref[...] = jnp.maximum(acc + bb_ref[...], 0.0)


def _kf(lu0, lu1, invden, g2u0, g2u1, invdeg, gftwt, w1t, w2t, gb, bb):
  blk = 1000
  grid = (N // blk,)
  big = pl.BlockSpec((blk, D), lambda i: (i, 0))
  col = pl.BlockSpec((blk, 1), lambda i: (i, 0))
  wgt = pl.BlockSpec((D, D), lambda i: (0, 0))
  row = pl.BlockSpec((1, D), lambda i: (0, 0))
  return pl.pallas_call(
      _kf_body,
      grid=grid,
      in_specs=[big, big, col, big, big, col, wgt, wgt, wgt, row, row],
      out_specs=big,
      out_shape=jax.ShapeDtypeStruct((N, D), f32),
  )(lu0, lu1, invden, g2u0, g2u1, invdeg, gftwt, w1t, w2t, gb, bb)


@jax.jit
def kernel(node_features, edge_index, linear_weights, attention_weights,
           wt_W, wt_b, gft_W, gft_b):
  wt = linear_weights.T
  a = jnp.reshape(attention_weights, (2, D)).T
  h, s12 = _k1(node_features, wt, a)

  lu, den, deg = _ka(s12[:, 0], s12[:, 1], edge_index, h)
  g1u = _kb(edge_index, node_features, den[0, :8])
  g1, invdeg, invden = _kc(g1u, den, deg)
  g2u = _kb(edge_index, g1, den[0, :8])

  out = _kf(lu[0], lu[1], invden.reshape(N, 1),
            g2u[0], g2u[1], invdeg.reshape(N, 1),
            gft_W.T, wt_W[:, :D].T, wt_W[:, D:].T,
            gft_b.reshape(1, D), wt_b.reshape(1, D))
  return out


# speedup vs baseline: 13.4632x; 1.0279x over previous
import functools

import jax
import jax.numpy as jnp
from jax import lax
from jax.experimental import pallas as pl
from jax.experimental.pallas import tpu as pltpu
from jax.experimental.pallas import tpu_sc as plsc

N = 10000
E = 320000
D = 128
NC = 2
NS = 16
NW = NC * NS
EPW = E // NW
C = 80
NCHUNK = EPW // C
GROUPS = C // 16
ZR = 125
RPT = N // NS
S1D = 624
RS = 320
SUB = 80

f32 = jnp.float32
i32 = jnp.int32

_mesh = plsc.VectorSubcoreMesh(
    core_axis_name="c", subcore_axis_name="s", num_cores=NC, num_subcores=NS)


def _zero_rows(ref, nrows):
  def body(r, carry):
    for j in range(D // 16):
      ref[r, pl.ds(j * 16, 16)] = jnp.zeros((16,), f32)
    return carry
  lax.fori_loop(0, nrows, body, 0)


def _fill_1d(ref, n, value):
  def body(g, carry):
    ref[pl.ds(g * 16, 16)] = jnp.full((16,), value, f32)
    return carry
  lax.fori_loop(0, n // 16, body, 0)


def _zero_acc_2d(acc_s, zbuf, sid):
  for k in range(RPT // ZR):
    pltpu.sync_copy(zbuf, acc_s.at[pl.ds(sid * RPT + k * ZR, ZR)])


def _copy_1d_striped(src_ref, dst_ref, sid):
  pltpu.sync_copy(src_ref.at[pl.ds(sid * S1D, S1D)],
                  dst_ref.at[pl.ds(sid * S1D, S1D)])
  @pl.when(sid == NS - 1)
  def _():
    pltpu.sync_copy(src_ref.at[pl.ds(NS * S1D, N - NS * S1D)],
                    dst_ref.at[pl.ds(NS * S1D, N - NS * S1D)])


def _zero_1d_striped(zvec, dst_ref, sid):
  pltpu.sync_copy(zvec.at[pl.ds(0, S1D)], dst_ref.at[pl.ds(sid * S1D, S1D)])
  @pl.when(sid == NS - 1)
  def _():
    pltpu.sync_copy(zvec.at[pl.ds(0, N - NS * S1D)],
                    dst_ref.at[pl.ds(NS * S1D, N - NS * S1D)])


def _k1_body(x_ref, wt_ref, a_ref, h_ref, s12_ref):
  h = jnp.dot(x_ref[...], wt_ref[...], preferred_element_type=f32)
  h_ref[...] = h
  s12_ref[...] = jnp.dot(h, a_ref[...], preferred_element_type=f32)


def _k1(x, wt, a):
  blk = 1000
  grid = (N // blk,)
  return pl.pallas_call(
      _k1_body,
      grid=grid,
      in_specs=[
          pl.BlockSpec((blk, D), lambda i: (i, 0)),
          pl.BlockSpec((D, D), lambda i: (0, 0)),
          pl.BlockSpec((D, 2), lambda i: (0, 0)),
      ],
      out_specs=[
          pl.BlockSpec((blk, D), lambda i: (i, 0)),
          pl.BlockSpec((blk, 2), lambda i: (i, 0)),
      ],
      out_shape=[
          jax.ShapeDtypeStruct((N, D), f32),
          jax.ShapeDtypeStruct((N, 2), f32),
      ],
  )(x, wt, a)


def _zero_acc_from_rows(acc_s, rows_v, sid):
  for k in range(RPT // C):
    pltpu.sync_copy(rows_v, acc_s.at[pl.ds(sid * RPT + k * C, C)])
  rem = RPT - (RPT // C) * C
  if rem:
    pltpu.sync_copy(rows_v.at[pl.ds(0, rem)],
                    acc_s.at[pl.ds(sid * RPT + (RPT // C) * C, rem)])


NSLOT = 3


def _ka_body(s1, s2, ei, h, lu_out, den_out, deg_out,
             srcA, srcB, srcC, dstA, dstB, dstC, rowsA, rowsB, rowsC,
             s1A, s1B, s1C, s2A, s2B, s2C, wA, wB, wC, ones_v, zvec,
             acc_s, den_s, deg_s,
             isemA, isemB, isemC, gsemA, gsemB, gsemC, ssemA, ssemB, ssemC):
  cid = lax.axis_index("c")
  sid = lax.axis_index("s")
  wid = sid * NC + cid
  base = wid * EPW

  _zero_rows(rowsA, C)
  _fill_1d(zvec, S1D + 16, 0.0)
  _fill_1d(ones_v, C, 1.0)
  _zero_acc_from_rows(acc_s, rowsA, sid)
  _zero_1d_striped(zvec, den_s, sid)
  _zero_1d_striped(zvec, deg_s, sid)
  plsc.subcore_barrier()

  slots = ((srcA, dstA, rowsA, s1A, s2A, wA, isemA, gsemA, ssemA),
           (srcB, dstB, rowsB, s1B, s2B, wB, isemB, gsemB, ssemB),
           (srcC, dstC, rowsC, s1C, s2C, wC, isemC, gsemC, ssemC))

  def phase1(c, k, slot):
    srcS, dstS, rowsS, s1S, s2S, wS, isem, gsem, ssem = slot
    @pl.when(k > 0)
    def _():
      pltpu.make_async_copy(rowsS, acc_s.at[dstS], ssem).wait()
    off = base + c * C
    pltpu.async_copy(ei.at[0, pl.ds(off, C)], srcS, isem)
    pltpu.async_copy(ei.at[1, pl.ds(off, C)], dstS, isem)

  def phase2(c, slot):
    srcS, dstS, rowsS, s1S, s2S, wS, isem, gsem, ssem = slot
    off = base + c * C
    pltpu.make_async_copy(ei.at[0, pl.ds(off, C)], srcS, isem).wait()
    pltpu.make_async_copy(ei.at[1, pl.ds(off, C)], dstS, isem).wait()
    pltpu.async_copy(h.at[srcS], rowsS, gsem)
    pltpu.async_copy(s1.at[srcS], s1S, gsem)
    pltpu.async_copy(s2.at[dstS], s2S, gsem)

  def phase3(c, slot):
    srcS, dstS, rowsS, s1S, s2S, wS, isem, gsem, ssem = slot
    pltpu.make_async_copy(h.at[srcS], rowsS, gsem).wait()
    pltpu.make_async_copy(s1.at[srcS], s1S, gsem).wait()
    pltpu.make_async_copy(s2.at[dstS], s2S, gsem).wait()

    def group_body(g, gcarry):
      bq = g * 16
      e = s1S[pl.ds(bq, 16)] + s2S[pl.ds(bq, 16)]
      e = jnp.where(e >= 0.0, e, 0.2 * e)
      w = jnp.exp(e)
      wS[pl.ds(bq, 16)] = w
      for i in range(16):
        r = bq + i
        wb = plsc.load_gather(wS, [jnp.full((16,), r, i32)])
        for j in range(D // 16):
          rows_slice = rowsS[r, pl.ds(j * 16, 16)]
          rowsS[r, pl.ds(j * 16, 16)] = rows_slice * wb
      return gcarry
    lax.fori_loop(0, GROUPS, group_body, 0)

    pltpu.async_copy(rowsS, acc_s.at[dstS], ssem, add=True)
    pltpu.sync_copy(wS, den_s.at[dstS], add=True)
    pltpu.sync_copy(ones_v, deg_s.at[dstS], add=True)

  def body3(k, carry):
    c0 = NSLOT * k
    for s in range(NSLOT):
      @pl.when(c0 + s < NCHUNK)
      def _(s=s):
        phase1(c0 + s, k, slots[s])
    for s in range(NSLOT):
      @pl.when(c0 + s < NCHUNK)
      def _(s=s):
        phase2(c0 + s, slots[s])
    for s in range(NSLOT):
      @pl.when(c0 + s < NCHUNK)
      def _(s=s):
        phase3(c0 + s, slots[s])
    return carry
  lax.fori_loop(0, (NCHUNK + NSLOT - 1) // NSLOT, body3, 0)
  for s in range(NSLOT):
    srcS, dstS, rowsS = slots[s][0], slots[s][1], slots[s][2]
    pltpu.make_async_copy(rowsS, acc_s.at[dstS], slots[s][8]).wait()

  plsc.subcore_barrier()
  for k in range(RPT // ZR):
    off = sid * RPT + k * ZR
    pltpu.sync_copy(acc_s.at[pl.ds(off, ZR)], lu_out.at[cid, pl.ds(off, ZR)])
  _copy_1d_striped(den_s, den_out.at[cid], sid)
  _copy_1d_striped(deg_s, deg_out.at[cid], sid)


_ka = functools.partial(
    pl.kernel,
    _ka_body,
    out_type=[
        jax.ShapeDtypeStruct((NC, N, D), f32),
        jax.ShapeDtypeStruct((NC, N), f32),
        jax.ShapeDtypeStruct((NC, N), f32),
    ],
    mesh=_mesh,
    compiler_params=pltpu.CompilerParams(use_tc_tiling_on_sc=False, needs_layout_passes=False),
    scratch_types=(
        [pltpu.VMEM((C,), i32) for _ in range(3)]
        + [pltpu.VMEM((C,), i32) for _ in range(3)]
        + [pltpu.VMEM((C, D), f32) for _ in range(3)]
        + [pltpu.VMEM((C,), f32) for _ in range(3)]
        + [pltpu.VMEM((C,), f32) for _ in range(3)]
        + [pltpu.VMEM((C,), f32) for _ in range(3)]
        + [
            pltpu.VMEM((C,), f32),
            pltpu.VMEM((S1D + 16,), f32),
            pltpu.VMEM_SHARED((N, D), f32),
            pltpu.VMEM_SHARED((N,), f32),
            pltpu.VMEM_SHARED((N,), f32),
        ]
        + [pltpu.SemaphoreType.DMA for _ in range(9)]
    ),
)()


def _kb_body(ei, table, tok, g_out,
             srcA, srcB, srcC, dstA, dstB, dstC, rowsA, rowsB, rowsC, acc_s,
             isemA, isemB, isemC, gsemA, gsemB, gsemC, ssemA, ssemB, ssemC):
  cid = lax.axis_index("c")
  sid = lax.axis_index("s")
  wid = sid * NC + cid
  base = wid * EPW

  _zero_rows(rowsA, C)
  _zero_acc_from_rows(acc_s, rowsA, sid)
  plsc.subcore_barrier()

  slots = ((srcA, dstA, rowsA, isemA, gsemA, ssemA),
           (srcB, dstB, rowsB, isemB, gsemB, ssemB),
           (srcC, dstC, rowsC, isemC, gsemC, ssemC))

  def phase1(c, k, slot):
    srcS, dstS, rowsS, isem, gsem, ssem = slot
    @pl.when(k > 0)
    def _():
      pltpu.make_async_copy(rowsS, acc_s.at[dstS], ssem).wait()
    off = base + c * C
    pltpu.async_copy(ei.at[0, pl.ds(off, C)], srcS, isem)
    pltpu.async_copy(ei.at[1, pl.ds(off, C)], dstS, isem)

  def phase2(c, slot):
    srcS, dstS, rowsS, isem, gsem, ssem = slot
    off = base + c * C
    pltpu.make_async_copy(ei.at[0, pl.ds(off, C)], srcS, isem).wait()
    pltpu.make_async_copy(ei.at[1, pl.ds(off, C)], dstS, isem).wait()
    pltpu.async_copy(table.at[srcS], rowsS, gsem)

  def phase3(c, slot):
    srcS, dstS, rowsS, isem, gsem, ssem = slot
    pltpu.make_async_copy(table.at[srcS], rowsS, gsem).wait()
    pltpu.async_copy(rowsS, acc_s.at[dstS], ssem, add=True)

  def body3(k, carry):
    c0 = NSLOT * k
    for s in range(NSLOT):
      @pl.when(c0 + s < NCHUNK)
      def _(s=s):
        phase1(c0 + s, k, slots[s])
    for s in range(NSLOT):
      @pl.when(c0 + s < NCHUNK)
      def _(s=s):
        phase2(c0 + s, slots[s])
    for s in range(NSLOT):
      @pl.when(c0 + s < NCHUNK)
      def _(s=s):
        phase3(c0 + s, slots[s])
    return carry
  lax.fori_loop(0, (NCHUNK + NSLOT - 1) // NSLOT, body3, 0)
  for s in range(NSLOT):
    srcS, dstS, rowsS = slots[s][0], slots[s][1], slots[s][2]
    pltpu.make_async_copy(rowsS, acc_s.at[dstS], slots[s][5]).wait()

  plsc.subcore_barrier()
  for k in range(RPT // ZR):
    off = sid * RPT + k * ZR
    pltpu.sync_copy(acc_s.at[pl.ds(off, ZR)], g_out.at[cid, pl.ds(off, ZR)])


_kb = functools.partial(
    pl.kernel,
    _kb_body,
    out_type=jax.ShapeDtypeStruct((NC, N, D), f32),
    mesh=_mesh,
    compiler_params=pltpu.CompilerParams(use_tc_tiling_on_sc=False, needs_layout_passes=False),
    scratch_types=(
        [pltpu.VMEM((C,), i32) for _ in range(3)]
        + [pltpu.VMEM((C,), i32) for _ in range(3)]
        + [pltpu.VMEM((C, D), f32) for _ in range(3)]
        + [pltpu.VMEM_SHARED((N, D), f32)]
        + [pltpu.SemaphoreType.DMA for _ in range(9)]
    ),
)()


def _kc_body(g1u, den, deg, g1_out, invdeg_out, invden_out,
             buf0, buf1, d0_v, d1_v, n0_v, n1_v, invdeg_v, invden_v, sem):
  cid = lax.axis_index("c")
  sid = lax.axis_index("s")
  wid = sid * NC + cid

  for k in range(RS // SUB):
    row0 = wid * RS + k * SUB

    @pl.when(row0 < N)
    def _():
      pltpu.sync_copy(g1u.at[0, pl.ds(row0, SUB)], buf0)
      pltpu.sync_copy(g1u.at[1, pl.ds(row0, SUB)], buf1)
      pltpu.sync_copy(deg.at[0, pl.ds(row0, SUB)], d0_v)
      pltpu.sync_copy(deg.at[1, pl.ds(row0, SUB)], d1_v)
      pltpu.sync_copy(den.at[0, pl.ds(row0, SUB)], n0_v)
      pltpu.sync_copy(den.at[1, pl.ds(row0, SUB)], n1_v)

      def vec_body(g, carry):
        b = g * 16
        dg = d0_v[pl.ds(b, 16)] + d1_v[pl.ds(b, 16)]
        dg = jnp.maximum(dg, 1.0)
        invdeg_v[pl.ds(b, 16)] = 1.0 / dg
        dn = n0_v[pl.ds(b, 16)] + n1_v[pl.ds(b, 16)] + 1e-16
        invden_v[pl.ds(b, 16)] = 1.0 / dn
        return carry
      lax.fori_loop(0, SUB // 16, vec_body, 0)

      def row_body(i, carry):
        wb = plsc.load_gather(invdeg_v, [jnp.full((16,), i, i32)])
        for j in range(D // 16):
          buf0[i, pl.ds(j * 16, 16)] = (
              buf0[i, pl.ds(j * 16, 16)] + buf1[i, pl.ds(j * 16, 16)]) * wb
        return carry
      lax.fori_loop(0, SUB, row_body, 0)

      pltpu.sync_copy(buf0, g1_out.at[pl.ds(row0, SUB)])
      pltpu.sync_copy(invdeg_v, invdeg_out.at[pl.ds(row0, SUB)])
      pltpu.sync_copy(invden_v, invden_out.at[pl.ds(row0, SUB)])


_kc = functools.partial(
    pl.kernel,
    _kc_body,
    out_type=[
        jax.ShapeDtypeStruct((N, D), f32),
        jax.ShapeDtypeStruct((N,), f32),
        jax.ShapeDtypeStruct((N,), f32),
    ],
    mesh=_mesh,
    compiler_params=pltpu.CompilerParams(use_tc_tiling_on_sc=False, needs_layout_passes=False),
    scratch_types=[
        pltpu.VMEM((SUB, D), f32),
        pltpu.VMEM((SUB, D), f32),
        pltpu.VMEM((SUB,), f32),
        pltpu.VMEM((SUB,), f32),
        pltpu.VMEM((SUB,), f32),
        pltpu.VMEM((SUB,), f32),
        pltpu.VMEM((SUB,), f32),
        pltpu.VMEM((SUB,), f32),
        pltpu.SemaphoreType.DMA,
    ],
)()


def _kf_body(lu0_ref, lu1_ref, invden_ref, g2u0_ref, g2u1_ref, invdeg_ref,
             gftwt_ref, w1t_ref, w2t_ref, gb_ref, bb_ref, out_ref):
  lu = (lu0_ref[...] + lu1_ref[...]) * invden_ref[...]
  local = jnp.where(lu > 0.0, lu, jnp.exp(jnp.minimum(lu, 0.0)) - 1.0)
  g2 = (g2u0_ref[...] + g2u1_ref[...]) * invdeg_ref[...]
  gf = jnp.dot(g2, gftwt_ref[...], preferred_element_type=f32) + gb_ref[...]
  acc = jnp.dot(local, w1t_ref[...], preferred_element_type=f32)
  acc = acc + jnp.dot(gf, w2t_ref[...], preferred_element_type=f32)
  out_ref[...] = jnp.maximum(acc + bb_ref[...], 0.0)


def _kf(lu0, lu1, invden, g2u0, g2u1, invdeg, gftwt, w1t, w2t, gb, bb):
  blk = 1000
  grid = (N // blk,)
  big = pl.BlockSpec((blk, D), lambda i: (i, 0))
  col = pl.BlockSpec((blk, 1), lambda i: (i, 0))
  wgt = pl.BlockSpec((D, D), lambda i: (0, 0))
  row = pl.BlockSpec((1, D), lambda i: (0, 0))
  return pl.pallas_call(
      _kf_body,
      grid=grid,
      in_specs=[big, big, col, big, big, col, wgt, wgt, wgt, row, row],
      out_specs=big,
      out_shape=jax.ShapeDtypeStruct((N, D), f32),
  )(lu0, lu1, invden, g2u0, g2u1, invdeg, gftwt, w1t, w2t, gb, bb)


@jax.jit
def kernel(node_features, edge_index, linear_weights, attention_weights,
           wt_W, wt_b, gft_W, gft_b):
  wt = linear_weights.T
  a = jnp.reshape(attention_weights, (2, D)).T
  h, s12 = _k1(node_features, wt, a)

  lu, den, deg = _ka(s12[:, 0], s12[:, 1], edge_index, h)
  g1u = _kb(edge_index, node_features, den[0, :8])
  g1, invdeg, invden = _kc(g1u, den, deg)
  g2u = _kb(edge_index, g1, den[0, :8])

  out = _kf(lu[0], lu[1], invden.reshape(N, 1),
            g2u[0], g2u[1], invdeg.reshape(N, 1),
            gft_W.T, wt_W[:, :D].T, wt_W[:, D:].T,
            gft_b.reshape(1, D), wt_b.reshape(1, D))
  return out


# speedup vs baseline: 15.6120x; 1.1596x over previous
import functools

import jax
import jax.numpy as jnp
from jax import lax
from jax.experimental import pallas as pl
from jax.experimental.pallas import tpu as pltpu
from jax.experimental.pallas import tpu_sc as plsc

N = 10000
E = 320000
D = 128
NC = 2
NS = 16
NW = NC * NS
EPW = E // NW
C = 80
NCHUNK = EPW // C
GROUPS = C // 16
ZR = 125
RPT = N // NS
S1D = 624
RS = 320
SUB = 80

f32 = jnp.float32
i32 = jnp.int32

_mesh = plsc.VectorSubcoreMesh(
    core_axis_name="c", subcore_axis_name="s", num_cores=NC, num_subcores=NS)


def _zero_rows(ref, nrows):
  def body(r, carry):
    for j in range(D // 16):
      ref[r, pl.ds(j * 16, 16)] = jnp.zeros((16,), f32)
    return carry
  lax.fori_loop(0, nrows, body, 0)


def _fill_1d(ref, n, value):
  def body(g, carry):
    ref[pl.ds(g * 16, 16)] = jnp.full((16,), value, f32)
    return carry
  lax.fori_loop(0, n // 16, body, 0)


def _zero_acc_2d(acc_s, zbuf, sid):
  for k in range(RPT // ZR):
    pltpu.sync_copy(zbuf, acc_s.at[pl.ds(sid * RPT + k * ZR, ZR)])


def _copy_1d_striped(src_ref, dst_ref, sid):
  pltpu.sync_copy(src_ref.at[pl.ds(sid * S1D, S1D)],
                  dst_ref.at[pl.ds(sid * S1D, S1D)])
  @pl.when(sid == NS - 1)
  def _():
    pltpu.sync_copy(src_ref.at[pl.ds(NS * S1D, N - NS * S1D)],
                    dst_ref.at[pl.ds(NS * S1D, N - NS * S1D)])


def _zero_1d_striped(zvec, dst_ref, sid):
  pltpu.sync_copy(zvec.at[pl.ds(0, S1D)], dst_ref.at[pl.ds(sid * S1D, S1D)])
  @pl.when(sid == NS - 1)
  def _():
    pltpu.sync_copy(zvec.at[pl.ds(0, N - NS * S1D)],
                    dst_ref.at[pl.ds(NS * S1D, N - NS * S1D)])


def _k1_body(x_ref, wt_ref, a_ref, h_ref, s12_ref):
  h = jnp.dot(x_ref[...], wt_ref[...], preferred_element_type=f32)
  h_ref[...] = h
  s12_ref[...] = jnp.dot(h, a_ref[...], preferred_element_type=f32)


def _k1(x, wt, a):
  blk = 1000
  grid = (N // blk,)
  return pl.pallas_call(
      _k1_body,
      grid=grid,
      in_specs=[
          pl.BlockSpec((blk, D), lambda i: (i, 0)),
          pl.BlockSpec((D, D), lambda i: (0, 0)),
          pl.BlockSpec((D, 2), lambda i: (0, 0)),
      ],
      out_specs=[
          pl.BlockSpec((blk, D), lambda i: (i, 0)),
          pl.BlockSpec((blk, 2), lambda i: (i, 0)),
      ],
      out_shape=[
          jax.ShapeDtypeStruct((N, D), f32),
          jax.ShapeDtypeStruct((N, 2), f32),
      ],
  )(x, wt, a)


def _zero_acc_from_rows(acc_s, rows_v, sid):
  for k in range(RPT // C):
    pltpu.sync_copy(rows_v, acc_s.at[pl.ds(sid * RPT + k * C, C)])
  rem = RPT - (RPT // C) * C
  if rem:
    pltpu.sync_copy(rows_v.at[pl.ds(0, rem)],
                    acc_s.at[pl.ds(sid * RPT + (RPT // C) * C, rem)])


NSLOT = 3


def _ka_body(s1, s2, ei, h, lu_out, den_out, deg_out,
             srcA, srcB, srcC, dstA, dstB, dstC, rowsA, rowsB, rowsC,
             s1A, s1B, s1C, s2A, s2B, s2C, wA, wB, wC, ones_v, zvec,
             acc_s, den_s, deg_s,
             isemA, isemB, isemC, gsemA, gsemB, gsemC, ssemA, ssemB, ssemC):
  cid = lax.axis_index("c")
  sid = lax.axis_index("s")
  wid = sid * NC + cid
  base = wid * EPW

  _zero_rows(rowsA, C)
  _fill_1d(zvec, S1D + 16, 0.0)
  _fill_1d(ones_v, C, 1.0)
  _zero_acc_from_rows(acc_s, rowsA, sid)
  _zero_1d_striped(zvec, den_s, sid)
  _zero_1d_striped(zvec, deg_s, sid)
  plsc.subcore_barrier()

  slots = ((srcA, dstA, rowsA, s1A, s2A, wA, isemA, gsemA, ssemA),
           (srcB, dstB, rowsB, s1B, s2B, wB, isemB, gsemB, ssemB),
           (srcC, dstC, rowsC, s1C, s2C, wC, isemC, gsemC, ssemC))

  def phase1(c, k, slot):
    srcS, dstS, rowsS, s1S, s2S, wS, isem, gsem, ssem = slot
    @pl.when(k > 0)
    def _():
      pltpu.make_async_copy(rowsS, acc_s.at[dstS], ssem).wait()
      pltpu.make_async_copy(wS, den_s.at[dstS], ssem).wait()
      pltpu.make_async_copy(ones_v, deg_s.at[dstS], ssem).wait()
    off = base + c * C
    pltpu.async_copy(ei.at[0, pl.ds(off, C)], srcS, isem)
    pltpu.async_copy(ei.at[1, pl.ds(off, C)], dstS, isem)

  def phase2(c, slot):
    srcS, dstS, rowsS, s1S, s2S, wS, isem, gsem, ssem = slot
    off = base + c * C
    pltpu.make_async_copy(ei.at[0, pl.ds(off, C)], srcS, isem).wait()
    pltpu.make_async_copy(ei.at[1, pl.ds(off, C)], dstS, isem).wait()
    pltpu.async_copy(h.at[srcS], rowsS, gsem)
    pltpu.async_copy(s1.at[srcS], s1S, gsem)
    pltpu.async_copy(s2.at[dstS], s2S, gsem)

  def phase3(c, slot):
    srcS, dstS, rowsS, s1S, s2S, wS, isem, gsem, ssem = slot
    pltpu.make_async_copy(h.at[srcS], rowsS, gsem).wait()
    pltpu.make_async_copy(s1.at[srcS], s1S, gsem).wait()
    pltpu.make_async_copy(s2.at[dstS], s2S, gsem).wait()

    def group_body(g, gcarry):
      bq = g * 16
      e = s1S[pl.ds(bq, 16)] + s2S[pl.ds(bq, 16)]
      e = jnp.where(e >= 0.0, e, 0.2 * e)
      w = jnp.exp(e)
      wS[pl.ds(bq, 16)] = w
      dn = lax.GatherDimensionNumbers(
          offset_dims=(), collapsed_slice_dims=(0,), start_index_map=(0,))
      for i in range(16):
        r = bq + i
        wb = lax.gather(w, jnp.full((16, 1), i, i32), dn, (1,),
                        mode=lax.GatherScatterMode.PROMISE_IN_BOUNDS)
        for j in range(D // 16):
          rows_slice = rowsS[r, pl.ds(j * 16, 16)]
          rowsS[r, pl.ds(j * 16, 16)] = rows_slice * wb
      return gcarry
    lax.fori_loop(0, GROUPS, group_body, 0)

    pltpu.async_copy(rowsS, acc_s.at[dstS], ssem, add=True)
    pltpu.async_copy(wS, den_s.at[dstS], ssem, add=True)
    pltpu.async_copy(ones_v, deg_s.at[dstS], ssem, add=True)

  def body3(k, carry):
    c0 = NSLOT * k
    for s in range(NSLOT):
      @pl.when(c0 + s < NCHUNK)
      def _(s=s):
        phase1(c0 + s, k, slots[s])
    for s in range(NSLOT):
      @pl.when(c0 + s < NCHUNK)
      def _(s=s):
        phase2(c0 + s, slots[s])
    for s in range(NSLOT):
      @pl.when(c0 + s < NCHUNK)
      def _(s=s):
        phase3(c0 + s, slots[s])
    return carry
  lax.fori_loop(0, (NCHUNK + NSLOT - 1) // NSLOT, body3, 0)
  for s in range(NSLOT):
    srcS, dstS, rowsS, wS = slots[s][0], slots[s][1], slots[s][2], slots[s][5]
    pltpu.make_async_copy(rowsS, acc_s.at[dstS], slots[s][8]).wait()
    pltpu.make_async_copy(wS, den_s.at[dstS], slots[s][8]).wait()
    pltpu.make_async_copy(ones_v, deg_s.at[dstS], slots[s][8]).wait()

  plsc.subcore_barrier()
  for k in range(RPT // ZR):
    off = sid * RPT + k * ZR
    pltpu.sync_copy(acc_s.at[pl.ds(off, ZR)], lu_out.at[cid, pl.ds(off, ZR)])
  _copy_1d_striped(den_s, den_out.at[cid], sid)
  _copy_1d_striped(deg_s, deg_out.at[cid], sid)


_ka = functools.partial(
    pl.kernel,
    _ka_body,
    out_type=[
        jax.ShapeDtypeStruct((NC, N, D), f32),
        jax.ShapeDtypeStruct((NC, N), f32),
        jax.ShapeDtypeStruct((NC, N), f32),
    ],
    mesh=_mesh,
    compiler_params=pltpu.CompilerParams(use_tc_tiling_on_sc=False, needs_layout_passes=False),
    scratch_types=(
        [pltpu.VMEM((C,), i32) for _ in range(3)]
        + [pltpu.VMEM((C,), i32) for _ in range(3)]
        + [pltpu.VMEM((C, D), f32) for _ in range(3)]
        + [pltpu.VMEM((C,), f32) for _ in range(3)]
        + [pltpu.VMEM((C,), f32) for _ in range(3)]
        + [pltpu.VMEM((C,), f32) for _ in range(3)]
        + [
            pltpu.VMEM((C,), f32),
            pltpu.VMEM((S1D + 16,), f32),
            pltpu.VMEM_SHARED((N, D), f32),
            pltpu.VMEM_SHARED((N,), f32),
            pltpu.VMEM_SHARED((N,), f32),
        ]
        + [pltpu.SemaphoreType.DMA for _ in range(9)]
    ),
)()


def _kb_body(ei, table, tok, g_out,
             srcA, srcB, srcC, dstA, dstB, dstC, rowsA, rowsB, rowsC, acc_s,
             isemA, isemB, isemC, gsemA, gsemB, gsemC, ssemA, ssemB, ssemC):
  cid = lax.axis_index("c")
  sid = lax.axis_index("s")
  wid = sid * NC + cid
  base = wid * EPW

  _zero_rows(rowsA, C)
  _zero_acc_from_rows(acc_s, rowsA, sid)
  plsc.subcore_barrier()

  slots = ((srcA, dstA, rowsA, isemA, gsemA, ssemA),
           (srcB, dstB, rowsB, isemB, gsemB, ssemB),
           (srcC, dstC, rowsC, isemC, gsemC, ssemC))

  def phase1(c, k, slot):
    srcS, dstS, rowsS, isem, gsem, ssem = slot
    @pl.when(k > 0)
    def _():
      pltpu.make_async_copy(rowsS, acc_s.at[dstS], ssem).wait()
    off = base + c * C
    pltpu.async_copy(ei.at[0, pl.ds(off, C)], srcS, isem)
    pltpu.async_copy(ei.at[1, pl.ds(off, C)], dstS, isem)

  def phase2(c, slot):
    srcS, dstS, rowsS, isem, gsem, ssem = slot
    off = base + c * C
    pltpu.make_async_copy(ei.at[0, pl.ds(off, C)], srcS, isem).wait()
    pltpu.make_async_copy(ei.at[1, pl.ds(off, C)], dstS, isem).wait()
    pltpu.async_copy(table.at[srcS], rowsS, gsem)

  def phase3(c, slot):
    srcS, dstS, rowsS, isem, gsem, ssem = slot
    pltpu.make_async_copy(table.at[srcS], rowsS, gsem).wait()
    pltpu.async_copy(rowsS, acc_s.at[dstS], ssem, add=True)

  def body3(k, carry):
    c0 = NSLOT * k
    for s in range(NSLOT):
      @pl.when(c0 + s < NCHUNK)
      def _(s=s):
        phase1(c0 + s, k, slots[s])
    for s in range(NSLOT):
      @pl.when(c0 + s < NCHUNK)
      def _(s=s):
        phase2(c0 + s, slots[s])
    for s in range(NSLOT):
      @pl.when(c0 + s < NCHUNK)
      def _(s=s):
        phase3(c0 + s, slots[s])
    return carry
  lax.fori_loop(0, (NCHUNK + NSLOT - 1) // NSLOT, body3, 0)
  for s in range(NSLOT):
    srcS, dstS, rowsS = slots[s][0], slots[s][1], slots[s][2]
    pltpu.make_async_copy(rowsS, acc_s.at[dstS], slots[s][5]).wait()

  plsc.subcore_barrier()
  for k in range(RPT // ZR):
    off = sid * RPT + k * ZR
    pltpu.sync_copy(acc_s.at[pl.ds(off, ZR)], g_out.at[cid, pl.ds(off, ZR)])


_kb = functools.partial(
    pl.kernel,
    _kb_body,
    out_type=jax.ShapeDtypeStruct((NC, N, D), f32),
    mesh=_mesh,
    compiler_params=pltpu.CompilerParams(use_tc_tiling_on_sc=False, needs_layout_passes=False),
    scratch_types=(
        [pltpu.VMEM((C,), i32) for _ in range(3)]
        + [pltpu.VMEM((C,), i32) for _ in range(3)]
        + [pltpu.VMEM((C, D), f32) for _ in range(3)]
        + [pltpu.VMEM_SHARED((N, D), f32)]
        + [pltpu.SemaphoreType.DMA for _ in range(9)]
    ),
)()


def _kc_body(g0_ref, g1_ref, degt_ref, out_ref):
  dg = degt_ref[...]
  invdeg = 1.0 / jnp.maximum(dg[:, 0:1] + dg[:, 1:2], 1.0)
  out_ref[...] = (g0_ref[...] + g1_ref[...]) * invdeg


def _kc(g0, g1, degt):
  blk = 1000
  grid = (N // blk,)
  big = pl.BlockSpec((blk, D), lambda i: (i, 0))
  two = pl.BlockSpec((blk, 2), lambda i: (i, 0))
  return pl.pallas_call(
      _kc_body,
      grid=grid,
      in_specs=[big, big, two],
      out_specs=big,
      out_shape=jax.ShapeDtypeStruct((N, D), f32),
  )(g0, g1, degt)


def _kf_body(lu0_ref, lu1_ref, dent_ref, g2u0_ref, g2u1_ref, degt_ref,
             gftwt_ref, w1t_ref, w2t_ref, gb_ref, bb_ref, out_ref):
  dn = dent_ref[...]
  invden = 1.0 / (dn[:, 0:1] + dn[:, 1:2] + 1e-16)
  dg = degt_ref[...]
  invdeg = 1.0 / jnp.maximum(dg[:, 0:1] + dg[:, 1:2], 1.0)
  lu = (lu0_ref[...] + lu1_ref[...]) * invden
  local = jnp.where(lu > 0.0, lu, jnp.exp(jnp.minimum(lu, 0.0)) - 1.0)
  g2 = (g2u0_ref[...] + g2u1_ref[...]) * invdeg
  gf = jnp.dot(g2, gftwt_ref[...], preferred_element_type=f32) + gb_ref[...]
  acc = jnp.dot(local, w1t_ref[...], preferred_element_type=f32)
  acc = acc + jnp.dot(gf, w2t_ref[...], preferred_element_type=f32)
  out_ref[...] = jnp.maximum(acc + bb_ref[...], 0.0)


def _kf(lu0, lu1, dent, g2u0, g2u1, degt, gftwt, w1t, w2t, gb, bb):
  blk = 1000
  grid = (N // blk,)
  big = pl.BlockSpec((blk, D), lambda i: (i, 0))
  two = pl.BlockSpec((blk, 2), lambda i: (i, 0))
  wgt = pl.BlockSpec((D, D), lambda i: (0, 0))
  row = pl.BlockSpec((1, D), lambda i: (0, 0))
  return pl.pallas_call(
      _kf_body,
      grid=grid,
      in_specs=[big, big, two, big, big, two, wgt, wgt, wgt, row, row],
      out_specs=big,
      out_shape=jax.ShapeDtypeStruct((N, D), f32),
  )(lu0, lu1, dent, g2u0, g2u1, degt, gftwt, w1t, w2t, gb, bb)


@jax.jit
def kernel(node_features, edge_index, linear_weights, attention_weights,
           wt_W, wt_b, gft_W, gft_b):
  wt = linear_weights.T
  a = jnp.reshape(attention_weights, (2, D)).T
  h, s12 = _k1(node_features, wt, a)

  lu, den, deg = _ka(s12[:, 0], s12[:, 1], edge_index, h)
  g1u = _kb(edge_index, node_features, den[0, :8])
  degt = deg.T
  dent = den.T
  g1 = _kc(g1u[0], g1u[1], degt)
  g2u = _kb(edge_index, g1, den[0, :8])

  out = _kf(lu[0], lu[1], dent, g2u[0], g2u[1], degt,
            gft_W.T, wt_W[:, :D].T, wt_W[:, D:].T,
            gft_b.reshape(1, D), wt_b.reshape(1, D))
  return out


# speedup vs baseline: 16.0183x; 1.0260x over previous
import functools

import jax
import jax.numpy as jnp
from jax import lax
from jax.experimental import pallas as pl
from jax.experimental.pallas import tpu as pltpu
from jax.experimental.pallas import tpu_sc as plsc

N = 10000
E = 320000
D = 128
NC = 2
NS = 16
NW = NC * NS
EPW = E // NW
C = 80
NCHUNK = EPW // C
GROUPS = C // 16
ZR = 125
RPT = N // NS
S1D = 624
RS = 320
SUB = 80

f32 = jnp.float32
i32 = jnp.int32

_mesh = plsc.VectorSubcoreMesh(
    core_axis_name="c", subcore_axis_name="s", num_cores=NC, num_subcores=NS)


def _zero_rows(ref, nrows):
  def body(r, carry):
    for j in range(D // 16):
      ref[r, pl.ds(j * 16, 16)] = jnp.zeros((16,), f32)
    return carry
  lax.fori_loop(0, nrows, body, 0)


def _fill_1d(ref, n, value):
  def body(g, carry):
    ref[pl.ds(g * 16, 16)] = jnp.full((16,), value, f32)
    return carry
  lax.fori_loop(0, n // 16, body, 0)


def _zero_acc_2d(acc_s, zbuf, sid):
  for k in range(RPT // ZR):
    pltpu.sync_copy(zbuf, acc_s.at[pl.ds(sid * RPT + k * ZR, ZR)])


def _copy_1d_striped(src_ref, dst_ref, sid):
  pltpu.sync_copy(src_ref.at[pl.ds(sid * S1D, S1D)],
                  dst_ref.at[pl.ds(sid * S1D, S1D)])
  @pl.when(sid == NS - 1)
  def _():
    pltpu.sync_copy(src_ref.at[pl.ds(NS * S1D, N - NS * S1D)],
                    dst_ref.at[pl.ds(NS * S1D, N - NS * S1D)])


def _zero_1d_striped(zvec, dst_ref, sid):
  pltpu.sync_copy(zvec.at[pl.ds(0, S1D)], dst_ref.at[pl.ds(sid * S1D, S1D)])
  @pl.when(sid == NS - 1)
  def _():
    pltpu.sync_copy(zvec.at[pl.ds(0, N - NS * S1D)],
                    dst_ref.at[pl.ds(NS * S1D, N - NS * S1D)])


def _k1_body(x_ref, wt_ref, a_ref, h_ref, s12_ref):
  h = jnp.dot(x_ref[...], wt_ref[...], preferred_element_type=f32)
  h_ref[...] = h
  s12_ref[...] = jnp.dot(h, a_ref[...], preferred_element_type=f32)


def _k1(x, wt, a):
  blk = 1000
  grid = (N // blk,)
  return pl.pallas_call(
      _k1_body,
      grid=grid,
      in_specs=[
          pl.BlockSpec((blk, D), lambda i: (i, 0)),
          pl.BlockSpec((D, D), lambda i: (0, 0)),
          pl.BlockSpec((D, 2), lambda i: (0, 0)),
      ],
      out_specs=[
          pl.BlockSpec((blk, D), lambda i: (i, 0)),
          pl.BlockSpec((blk, 2), lambda i: (i, 0)),
      ],
      out_shape=[
          jax.ShapeDtypeStruct((N, D), f32),
          jax.ShapeDtypeStruct((N, 2), f32),
      ],
  )(x, wt, a)


def _zero_acc_from_rows(acc_s, rows_v, sid):
  for k in range(RPT // C):
    pltpu.sync_copy(rows_v, acc_s.at[pl.ds(sid * RPT + k * C, C)])
  rem = RPT - (RPT // C) * C
  if rem:
    pltpu.sync_copy(rows_v.at[pl.ds(0, rem)],
                    acc_s.at[pl.ds(sid * RPT + (RPT // C) * C, rem)])


NSLOT = 4


def _ka_body(s1, s2, ei, h, lu_out, den_out, deg_out,
             srcA, srcB, srcC, srcD, dstA, dstB, dstC, dstD,
             rowsA, rowsB, rowsC, rowsD,
             s1A, s1B, s1C, s1D, s2A, s2B, s2C, s2D,
             wA, wB, wC, wD, ones_v, zvec,
             acc_s, den_s, deg_s,
             isemA, isemB, isemC, isemD, gsemA, gsemB, gsemC, gsemD,
             ssemA, ssemB, ssemC, ssemD):
  cid = lax.axis_index("c")
  sid = lax.axis_index("s")
  wid = sid * NC + cid
  base = wid * EPW

  _zero_rows(rowsA, C)
  _fill_1d(zvec, S1D + 16, 0.0)
  _fill_1d(ones_v, C, 1.0)
  _zero_acc_from_rows(acc_s, rowsA, sid)
  _zero_1d_striped(zvec, den_s, sid)
  _zero_1d_striped(zvec, deg_s, sid)
  plsc.subcore_barrier()

  slots = ((srcA, dstA, rowsA, s1A, s2A, wA, isemA, gsemA, ssemA),
           (srcB, dstB, rowsB, s1B, s2B, wB, isemB, gsemB, ssemB),
           (srcC, dstC, rowsC, s1C, s2C, wC, isemC, gsemC, ssemC),
           (srcD, dstD, rowsD, s1D, s2D, wD, isemD, gsemD, ssemD))

  def phase1(c, k, slot):
    srcS, dstS, rowsS, s1S, s2S, wS, isem, gsem, ssem = slot
    @pl.when(k > 0)
    def _():
      pltpu.make_async_copy(rowsS, acc_s.at[dstS], ssem).wait()
      pltpu.make_async_copy(wS, den_s.at[dstS], ssem).wait()
      pltpu.make_async_copy(ones_v, deg_s.at[dstS], ssem).wait()
    off = base + c * C
    pltpu.async_copy(ei.at[0, pl.ds(off, C)], srcS, isem)
    pltpu.async_copy(ei.at[1, pl.ds(off, C)], dstS, isem)

  def phase2(c, slot):
    srcS, dstS, rowsS, s1S, s2S, wS, isem, gsem, ssem = slot
    off = base + c * C
    pltpu.make_async_copy(ei.at[0, pl.ds(off, C)], srcS, isem).wait()
    pltpu.make_async_copy(ei.at[1, pl.ds(off, C)], dstS, isem).wait()
    pltpu.async_copy(h.at[srcS], rowsS, gsem)
    pltpu.async_copy(s1.at[srcS], s1S, gsem)
    pltpu.async_copy(s2.at[dstS], s2S, gsem)

  def phase3(c, slot):
    srcS, dstS, rowsS, s1S, s2S, wS, isem, gsem, ssem = slot
    pltpu.make_async_copy(h.at[srcS], rowsS, gsem).wait()
    pltpu.make_async_copy(s1.at[srcS], s1S, gsem).wait()
    pltpu.make_async_copy(s2.at[dstS], s2S, gsem).wait()

    def group_body(g, gcarry):
      bq = g * 16
      e = s1S[pl.ds(bq, 16)] + s2S[pl.ds(bq, 16)]
      e = jnp.where(e >= 0.0, e, 0.2 * e)
      w = jnp.exp(e)
      wS[pl.ds(bq, 16)] = w
      dn = lax.GatherDimensionNumbers(
          offset_dims=(), collapsed_slice_dims=(0,), start_index_map=(0,))
      for i in range(16):
        r = bq + i
        wb = lax.gather(w, jnp.full((16, 1), i, i32), dn, (1,),
                        mode=lax.GatherScatterMode.PROMISE_IN_BOUNDS)
        for j in range(D // 16):
          rows_slice = rowsS[r, pl.ds(j * 16, 16)]
          rowsS[r, pl.ds(j * 16, 16)] = rows_slice * wb
      return gcarry
    lax.fori_loop(0, GROUPS, group_body, 0)

    pltpu.async_copy(rowsS, acc_s.at[dstS], ssem, add=True)
    pltpu.async_copy(wS, den_s.at[dstS], ssem, add=True)
    pltpu.async_copy(ones_v, deg_s.at[dstS], ssem, add=True)

  def body3(k, carry):
    c0 = NSLOT * k
    for s in range(NSLOT):
      @pl.when(c0 + s < NCHUNK)
      def _(s=s):
        phase1(c0 + s, k, slots[s])
    for s in range(NSLOT):
      @pl.when(c0 + s < NCHUNK)
      def _(s=s):
        phase2(c0 + s, slots[s])
    for s in range(NSLOT):
      @pl.when(c0 + s < NCHUNK)
      def _(s=s):
        phase3(c0 + s, slots[s])
    return carry
  lax.fori_loop(0, (NCHUNK + NSLOT - 1) // NSLOT, body3, 0)
  for s in range(NSLOT):
    srcS, dstS, rowsS, wS = slots[s][0], slots[s][1], slots[s][2], slots[s][5]
    pltpu.make_async_copy(rowsS, acc_s.at[dstS], slots[s][8]).wait()
    pltpu.make_async_copy(wS, den_s.at[dstS], slots[s][8]).wait()
    pltpu.make_async_copy(ones_v, deg_s.at[dstS], slots[s][8]).wait()

  plsc.subcore_barrier()
  for k in range(RPT // ZR):
    off = sid * RPT + k * ZR
    pltpu.sync_copy(acc_s.at[pl.ds(off, ZR)], lu_out.at[cid, pl.ds(off, ZR)])
  _copy_1d_striped(den_s, den_out.at[cid], sid)
  _copy_1d_striped(deg_s, deg_out.at[cid], sid)


_ka = functools.partial(
    pl.kernel,
    _ka_body,
    out_type=[
        jax.ShapeDtypeStruct((NC, N, D), f32),
        jax.ShapeDtypeStruct((NC, N), f32),
        jax.ShapeDtypeStruct((NC, N), f32),
    ],
    mesh=_mesh,
    compiler_params=pltpu.CompilerParams(use_tc_tiling_on_sc=False, needs_layout_passes=False),
    scratch_types=(
        [pltpu.VMEM((C,), i32) for _ in range(NSLOT)]
        + [pltpu.VMEM((C,), i32) for _ in range(NSLOT)]
        + [pltpu.VMEM((C, D), f32) for _ in range(NSLOT)]
        + [pltpu.VMEM((C,), f32) for _ in range(NSLOT)]
        + [pltpu.VMEM((C,), f32) for _ in range(NSLOT)]
        + [pltpu.VMEM((C,), f32) for _ in range(NSLOT)]
        + [
            pltpu.VMEM((C,), f32),
            pltpu.VMEM((S1D + 16,), f32),
            pltpu.VMEM_SHARED((N, D), f32),
            pltpu.VMEM_SHARED((N,), f32),
            pltpu.VMEM_SHARED((N,), f32),
        ]
        + [pltpu.SemaphoreType.DMA for _ in range(3 * NSLOT)]
    ),
)()


def _kb_body(ei, table, tok, g_out,
             srcA, srcB, srcC, srcD, dstA, dstB, dstC, dstD,
             rowsA, rowsB, rowsC, rowsD, acc_s,
             isemA, isemB, isemC, isemD, gsemA, gsemB, gsemC, gsemD,
             ssemA, ssemB, ssemC, ssemD):
  cid = lax.axis_index("c")
  sid = lax.axis_index("s")
  wid = sid * NC + cid
  base = wid * EPW

  _zero_rows(rowsA, C)
  _zero_acc_from_rows(acc_s, rowsA, sid)
  plsc.subcore_barrier()

  slots = ((srcA, dstA, rowsA, isemA, gsemA, ssemA),
           (srcB, dstB, rowsB, isemB, gsemB, ssemB),
           (srcC, dstC, rowsC, isemC, gsemC, ssemC),
           (srcD, dstD, rowsD, isemD, gsemD, ssemD))

  def phase1(c, k, slot):
    srcS, dstS, rowsS, isem, gsem, ssem = slot
    @pl.when(k > 0)
    def _():
      pltpu.make_async_copy(rowsS, acc_s.at[dstS], ssem).wait()
    off = base + c * C
    pltpu.async_copy(ei.at[0, pl.ds(off, C)], srcS, isem)
    pltpu.async_copy(ei.at[1, pl.ds(off, C)], dstS, isem)

  def phase2(c, slot):
    srcS, dstS, rowsS, isem, gsem, ssem = slot
    off = base + c * C
    pltpu.make_async_copy(ei.at[0, pl.ds(off, C)], srcS, isem).wait()
    pltpu.make_async_copy(ei.at[1, pl.ds(off, C)], dstS, isem).wait()
    pltpu.async_copy(table.at[srcS], rowsS, gsem)

  def phase3(c, slot):
    srcS, dstS, rowsS, isem, gsem, ssem = slot
    pltpu.make_async_copy(table.at[srcS], rowsS, gsem).wait()
    pltpu.async_copy(rowsS, acc_s.at[dstS], ssem, add=True)

  def body3(k, carry):
    c0 = NSLOT * k
    for s in range(NSLOT):
      @pl.when(c0 + s < NCHUNK)
      def _(s=s):
        phase1(c0 + s, k, slots[s])
    for s in range(NSLOT):
      @pl.when(c0 + s < NCHUNK)
      def _(s=s):
        phase2(c0 + s, slots[s])
    for s in range(NSLOT):
      @pl.when(c0 + s < NCHUNK)
      def _(s=s):
        phase3(c0 + s, slots[s])
    return carry
  lax.fori_loop(0, (NCHUNK + NSLOT - 1) // NSLOT, body3, 0)
  for s in range(NSLOT):
    srcS, dstS, rowsS = slots[s][0], slots[s][1], slots[s][2]
    pltpu.make_async_copy(rowsS, acc_s.at[dstS], slots[s][5]).wait()

  plsc.subcore_barrier()
  for k in range(RPT // ZR):
    off = sid * RPT + k * ZR
    pltpu.sync_copy(acc_s.at[pl.ds(off, ZR)], g_out.at[cid, pl.ds(off, ZR)])


_kb = functools.partial(
    pl.kernel,
    _kb_body,
    out_type=jax.ShapeDtypeStruct((NC, N, D), f32),
    mesh=_mesh,
    compiler_params=pltpu.CompilerParams(use_tc_tiling_on_sc=False, needs_layout_passes=False),
    scratch_types=(
        [pltpu.VMEM((C,), i32) for _ in range(NSLOT)]
        + [pltpu.VMEM((C,), i32) for _ in range(NSLOT)]
        + [pltpu.VMEM((C, D), f32) for _ in range(NSLOT)]
        + [pltpu.VMEM_SHARED((N, D), f32)]
        + [pltpu.SemaphoreType.DMA for _ in range(3 * NSLOT)]
    ),
)()


def _kc_body(g0_ref, g1_ref, degt_ref, out_ref):
  dg = degt_ref[...]
  invdeg = 1.0 / jnp.maximum(dg[:, 0:1] + dg[:, 1:2], 1.0)
  out_ref[...] = (g0_ref[...] + g1_ref[...]) * invdeg


def _kc(g0, g1, degt):
  blk = 1000
  grid = (N // blk,)
  big = pl.BlockSpec((blk, D), lambda i: (i, 0))
  two = pl.BlockSpec((blk, 2), lambda i: (i, 0))
  return pl.pallas_call(
      _kc_body,
      grid=grid,
      in_specs=[big, big, two],
      out_specs=big,
      out_shape=jax.ShapeDtypeStruct((N, D), f32),
  )(g0, g1, degt)


def _kf_body(lu0_ref, lu1_ref, dent_ref, g2u0_ref, g2u1_ref, degt_ref,
             gftwt_ref, w1t_ref, w2t_ref, gb_ref, bb_ref, out_ref):
  dn = dent_ref[...]
  invden = 1.0 / (dn[:, 0:1] + dn[:, 1:2] + 1e-16)
  dg = degt_ref[...]
  invdeg = 1.0 / jnp.maximum(dg[:, 0:1] + dg[:, 1:2], 1.0)
  lu = (lu0_ref[...] + lu1_ref[...]) * invden
  local = jnp.where(lu > 0.0, lu, jnp.exp(jnp.minimum(lu, 0.0)) - 1.0)
  g2 = (g2u0_ref[...] + g2u1_ref[...]) * invdeg
  gf = jnp.dot(g2, gftwt_ref[...], preferred_element_type=f32) + gb_ref[...]
  acc = jnp.dot(local, w1t_ref[...], preferred_element_type=f32)
  acc = acc + jnp.dot(gf, w2t_ref[...], preferred_element_type=f32)
  out_ref[...] = jnp.maximum(acc + bb_ref[...], 0.0)


def _kf(lu0, lu1, dent, g2u0, g2u1, degt, gftwt, w1t, w2t, gb, bb):
  blk = 1000
  grid = (N // blk,)
  big = pl.BlockSpec((blk, D), lambda i: (i, 0))
  two = pl.BlockSpec((blk, 2), lambda i: (i, 0))
  wgt = pl.BlockSpec((D, D), lambda i: (0, 0))
  row = pl.BlockSpec((1, D), lambda i: (0, 0))
  return pl.pallas_call(
      _kf_body,
      grid=grid,
      in_specs=[big, big, two, big, big, two, wgt, wgt, wgt, row, row],
      out_specs=big,
      out_shape=jax.ShapeDtypeStruct((N, D), f32),
  )(lu0, lu1, dent, g2u0, g2u1, degt, gftwt, w1t, w2t, gb, bb)


@jax.jit
def kernel(node_features, edge_index, linear_weights, attention_weights,
           wt_W, wt_b, gft_W, gft_b):
  wt = linear_weights.T
  a = jnp.reshape(attention_weights, (2, D)).T
  h, s12 = _k1(node_features, wt, a)

  lu, den, deg = _ka(s12[:, 0], s12[:, 1], edge_index, h)
  g1u = _kb(edge_index, node_features, den[0, :8])
  degt = deg.T
  dent = den.T
  g1 = _kc(g1u[0], g1u[1], degt)
  g2u = _kb(edge_index, g1, den[0, :8])

  out = _kf(lu[0], lu[1], dent, g2u[0], g2u[1], degt,
            gft_W.T, wt_W[:, :D].T, wt_W[:, D:].T,
            gft_b.reshape(1, D), wt_b.reshape(1, D))
  return out
